# baseline (speedup 1.0000x reference)
import functools

import jax
import jax.numpy as jnp
from jax import lax
from jax.experimental import pallas as pl
from jax.experimental.pallas import tpu as pltpu
from jax.experimental.pallas import tpu_sc as plsc

QP_NUM = 64
CHANNEL = 192
B, H, W = 32, 64, 64

_WORKERS = 4
_PER = B // _WORKERS
_CPAD = 256


def _sc_gather(h2, b2, a2, idx):
  mesh = plsc.VectorSubcoreMesh(core_axis_name="c", subcore_axis_name="s")

  @functools.partial(
      pl.kernel,
      mesh=mesh,
      out_type=[jax.ShapeDtypeStruct((B, _CPAD), jnp.float32)] * 3,
      scratch_types=[
          pltpu.VMEM((_PER,), jnp.int32),
          pltpu.VMEM((_PER, _CPAD), jnp.float32),
          pltpu.VMEM((_PER, _CPAD), jnp.float32),
          pltpu.VMEM((_PER, _CPAD), jnp.float32),
          pltpu.SemaphoreType.DMA,
      ],
  )
  def k(h_hbm, b_hbm, a_hbm, idx_hbm, oh, ob, oa, idx_v, rh, rb, ra, sem):
    wid = lax.axis_index("s") * 2 + lax.axis_index("c")

    @pl.when(wid < _WORKERS)
    def _():
      base = wid * _PER
      pltpu.sync_copy(idx_hbm.at[pl.ds(base, _PER)], idx_v)
      pltpu.async_copy(h_hbm.at[idx_v], rh, sem).wait()
      pltpu.async_copy(b_hbm.at[idx_v], rb, sem).wait()
      pltpu.async_copy(a_hbm.at[idx_v], ra, sem).wait()
      pltpu.sync_copy(rh, oh.at[pl.ds(base, _PER)])
      pltpu.sync_copy(rb, ob.at[pl.ds(base, _PER)])
      pltpu.sync_copy(ra, oa.at[pl.ds(base, _PER)])

  return k(h2, b2, a2, idx)


_CBLK = 96
_NCH = CHANNEL // _CBLK
_NCHUNK = B * _NCH
_DEPTH = 6


def _tc_body(h_ref, b_ref, a_ref, x_hbm, o_hbm, in_buf, out_buf,
             in_sem, out_sem):
  n = pl.program_id(0)
  slot = lax.rem(n, _DEPTH)

  def chunk_slice(ref, m):
    return ref.at[m // _NCH, pl.ds((m % _NCH) * _CBLK, _CBLK)]

  @pl.when(n == 0)
  def _():
    for k in range(_DEPTH):
      pltpu.make_async_copy(
          chunk_slice(x_hbm, k), in_buf.at[k], in_sem.at[k]).start()

  @pl.when(n >= _DEPTH)
  def _():
    pltpu.make_async_copy(
        out_buf.at[slot], chunk_slice(o_hbm, n - _DEPTH), out_sem.at[slot]
    ).wait()

  pltpu.make_async_copy(
      chunk_slice(x_hbm, n), in_buf.at[slot], in_sem.at[slot]).wait()

  s = jax.nn.softplus(h_ref[...]).reshape(_CBLK, 1, 1)
  t = jnp.tanh(a_ref[...]).reshape(_CBLK, 1, 1)
  y = in_buf[slot] * s + b_ref[...].reshape(_CBLK, 1, 1)
  out_buf[slot] = y + jnp.tanh(y) * t

  pltpu.make_async_copy(
      out_buf.at[slot], chunk_slice(o_hbm, n), out_sem.at[slot]).start()

  @pl.when(n + _DEPTH < _NCHUNK)
  def _():
    pltpu.make_async_copy(
        chunk_slice(x_hbm, n + _DEPTH), in_buf.at[slot], in_sem.at[slot]
    ).start()

  @pl.when(n == _NCHUNK - 1)
  def _():
    for k in range(_DEPTH):
      m = _NCHUNK - _DEPTH + k
      pltpu.make_async_copy(
          out_buf.at[m % _DEPTH], chunk_slice(o_hbm, m),
          out_sem.at[m % _DEPTH]).wait()


def _tc_apply(x, hs, bs, as_):
  par_spec = pl.BlockSpec((1, _CBLK, 1, 1),
                          lambda n: (n // _NCH, n % _NCH, 0, 0))
  return pl.pallas_call(
      _tc_body,
      grid=(_NCHUNK,),
      in_specs=[
          par_spec, par_spec, par_spec,
          pl.BlockSpec(memory_space=pltpu.MemorySpace.HBM),
      ],
      out_specs=pl.BlockSpec(memory_space=pltpu.MemorySpace.HBM),
      out_shape=jax.ShapeDtypeStruct((B, CHANNEL, H, W), jnp.float32),
      scratch_shapes=[
          pltpu.VMEM((_DEPTH, _CBLK, H, W), jnp.float32),
          pltpu.VMEM((_DEPTH, _CBLK, H, W), jnp.float32),
          pltpu.SemaphoreType.DMA((_DEPTH,)),
          pltpu.SemaphoreType.DMA((_DEPTH,)),
      ],
  )(hs, bs, as_, x)


def _pad_table(t):
  t2 = t.reshape(QP_NUM, CHANNEL)
  return jnp.pad(t2, ((0, 0), (0, _CPAD - CHANNEL)))


@jax.jit
def kernel(x, index, h, b, a):
  idx = index.astype(jnp.int32)
  hs, bs, as_ = _sc_gather(_pad_table(h), _pad_table(b), _pad_table(a), idx)
  return _tc_apply(
      x,
      hs[:, :CHANNEL].reshape(B, CHANNEL, 1, 1),
      bs[:, :CHANNEL].reshape(B, CHANNEL, 1, 1),
      as_[:, :CHANNEL].reshape(B, CHANNEL, 1, 1),
  )


# speedup vs baseline: 1.4452x; 1.4452x over previous
import functools

import jax
import jax.numpy as jnp
from jax import lax
from jax.experimental import pallas as pl
from jax.experimental.pallas import tpu as pltpu
from jax.experimental.pallas import tpu_sc as plsc

QP_NUM = 64
CHANNEL = 192
B, H, W = 32, 64, 64

_WORKERS = 4
_PER = B // _WORKERS
_CPAD = 256


def _sc_gather(h2, b2, a2, idx):
  mesh = plsc.VectorSubcoreMesh(core_axis_name="c", subcore_axis_name="s")

  @functools.partial(
      pl.kernel,
      mesh=mesh,
      out_type=[jax.ShapeDtypeStruct((B, _CPAD), jnp.float32)] * 3,
      scratch_types=[
          pltpu.VMEM((_PER,), jnp.int32),
          pltpu.VMEM((_PER, _CPAD), jnp.float32),
          pltpu.VMEM((_PER, _CPAD), jnp.float32),
          pltpu.VMEM((_PER, _CPAD), jnp.float32),
          pltpu.SemaphoreType.DMA,
      ],
  )
  def k(h_hbm, b_hbm, a_hbm, idx_hbm, oh, ob, oa, idx_v, rh, rb, ra, sem):
    wid = lax.axis_index("s") * 2 + lax.axis_index("c")

    @pl.when(wid < _WORKERS)
    def _():
      base = wid * _PER
      pltpu.sync_copy(idx_hbm.at[pl.ds(base, _PER)], idx_v)
      pltpu.async_copy(h_hbm.at[idx_v], rh, sem).wait()
      pltpu.async_copy(b_hbm.at[idx_v], rb, sem).wait()
      pltpu.async_copy(a_hbm.at[idx_v], ra, sem).wait()
      pltpu.sync_copy(rh, oh.at[pl.ds(base, _PER)])
      pltpu.sync_copy(rb, ob.at[pl.ds(base, _PER)])
      pltpu.sync_copy(ra, oa.at[pl.ds(base, _PER)])

  return k(h2, b2, a2, idx)


_CBLK = 96
_NCH = CHANNEL // _CBLK
_NCHUNK = B * _NCH
_DEPTH = 6


def _tc_body(h_ref, b_ref, a_ref, x_hbm, o_hbm, in_buf, out_buf,
             in_sem, out_sem):
  n = pl.program_id(0)
  slot = lax.rem(n, _DEPTH)

  def chunk_slice(ref, m):
    return ref.at[m // _NCH, pl.ds((m % _NCH) * _CBLK, _CBLK)]

  @pl.when(n == 0)
  def _():
    for k in range(_DEPTH):
      pltpu.make_async_copy(
          chunk_slice(x_hbm, k), in_buf.at[k], in_sem.at[k]).start()

  @pl.when(n >= _DEPTH)
  def _():
    pltpu.make_async_copy(
        out_buf.at[slot], chunk_slice(o_hbm, n - _DEPTH), out_sem.at[slot]
    ).wait()

  pltpu.make_async_copy(
      chunk_slice(x_hbm, n), in_buf.at[slot], in_sem.at[slot]).wait()

  s = jax.nn.softplus(h_ref[...]).reshape(_CBLK, 1, 1)
  t = jnp.tanh(a_ref[...]).reshape(_CBLK, 1, 1)
  y = in_buf[slot] * s + b_ref[...].reshape(_CBLK, 1, 1)
  out_buf[slot] = y + jnp.tanh(y) * t

  pltpu.make_async_copy(
      out_buf.at[slot], chunk_slice(o_hbm, n), out_sem.at[slot]).start()

  @pl.when(n + _DEPTH < _NCHUNK)
  def _():
    pltpu.make_async_copy(
        chunk_slice(x_hbm, n + _DEPTH), in_buf.at[slot], in_sem.at[slot]
    ).start()

  @pl.when(n == _NCHUNK - 1)
  def _():
    for k in range(_DEPTH):
      m = _NCHUNK - _DEPTH + k
      pltpu.make_async_copy(
          out_buf.at[m % _DEPTH], chunk_slice(o_hbm, m),
          out_sem.at[m % _DEPTH]).wait()


_HF = (H * W) // 128


def _tc_apply(x, hs, bs, as_):
  x4 = x.reshape(B, CHANNEL, _HF, 128)
  par_spec = pl.BlockSpec((1, _CBLK, 1, 1),
                          lambda n: (n // _NCH, n % _NCH, 0, 0))
  out = pl.pallas_call(
      _tc_body,
      grid=(_NCHUNK,),
      in_specs=[
          par_spec, par_spec, par_spec,
          pl.BlockSpec(memory_space=pltpu.MemorySpace.HBM),
      ],
      out_specs=pl.BlockSpec(memory_space=pltpu.MemorySpace.HBM),
      out_shape=jax.ShapeDtypeStruct((B, CHANNEL, _HF, 128), jnp.float32),
      scratch_shapes=[
          pltpu.VMEM((_DEPTH, _CBLK, _HF, 128), jnp.float32),
          pltpu.VMEM((_DEPTH, _CBLK, _HF, 128), jnp.float32),
          pltpu.SemaphoreType.DMA((_DEPTH,)),
          pltpu.SemaphoreType.DMA((_DEPTH,)),
      ],
  )(hs, bs, as_, x4)
  return out.reshape(B, CHANNEL, H, W)


def _pad_table(t):
  t2 = t.reshape(QP_NUM, CHANNEL)
  return jnp.pad(t2, ((0, 0), (0, _CPAD - CHANNEL)))


@jax.jit
def kernel(x, index, h, b, a):
  idx = index.astype(jnp.int32)
  hs, bs, as_ = _sc_gather(_pad_table(h), _pad_table(b), _pad_table(a), idx)
  return _tc_apply(
      x,
      hs[:, :CHANNEL].reshape(B, CHANNEL, 1, 1),
      bs[:, :CHANNEL].reshape(B, CHANNEL, 1, 1),
      as_[:, :CHANNEL].reshape(B, CHANNEL, 1, 1),
  )
